# Initial kernel scaffold; baseline (speedup 1.0000x reference)
import functools

import jax
import jax.numpy as jnp
from jax import lax
from jax.experimental import pallas as pl
from jax.experimental.pallas import tpu as pltpu
from jax.experimental.pallas import tpu_sc as plsc

N = 10000
NP = 10240
D = 128
E = 320000
EPS = 1e-5

NC = 2
NS = 16
NW = NC * NS
EPW = E // NW
CH = 80
NCH = EPW // CH
RPT = NP // NS

_mesh = plsc.VectorSubcoreMesh(core_axis_name="c", subcore_axis_name="s")


def _zero_f32_buf(ref, rows, cols):
    @pl.loop(0, rows)
    def _(i):
        for j in range(cols // 16):
            ref[i, pl.ds(j * 16, 16)] = jnp.zeros((16,), jnp.float32)


@functools.partial(
    pl.kernel,
    out_type=jax.ShapeDtypeStruct((NC, NP, 16), jnp.float32),
    mesh=_mesh,
    scratch_types=[
        pltpu.VMEM((CH,), jnp.int32),
        pltpu.VMEM((CH, 16), jnp.float32),
        pltpu.VMEM((RPT, 16), jnp.float32),
        pltpu.VMEM_SHARED((NP, 16), jnp.float32),
    ],
)
def _sc_deg(dst_hbm, out_hbm, didx_v, ones_v, zbuf_v, deg_sh):
    c = lax.axis_index("c")
    s = lax.axis_index("s")
    wid = c * NS + s

    @pl.loop(0, CH)
    def _(i):
        ones_v[i, pl.ds(0, 16)] = jnp.ones((16,), jnp.float32)

    _zero_f32_buf(zbuf_v, RPT, 16)
    pltpu.sync_copy(zbuf_v, deg_sh.at[pl.ds(s * RPT, RPT)])
    plsc.subcore_barrier()

    @pl.loop(0, NCH)
    def _(j):
        base = wid * EPW + j * CH
        pltpu.sync_copy(dst_hbm.at[pl.ds(base, CH)], didx_v)
        pltpu.sync_copy(ones_v, deg_sh.at[didx_v], add=True)

    plsc.subcore_barrier()
    pltpu.sync_copy(deg_sh.at[pl.ds(s * RPT, RPT)],
                    out_hbm.at[c, pl.ds(s * RPT, RPT)])


@functools.partial(
    pl.kernel,
    out_type=jax.ShapeDtypeStruct((NC, NP, D), jnp.float32),
    mesh=_mesh,
    scratch_types=[
        pltpu.VMEM((CH,), jnp.int32),
        pltpu.VMEM((CH,), jnp.int32),
        pltpu.VMEM((CH, D), jnp.float32),
        pltpu.VMEM_SHARED((NP, D), jnp.float32),
        pltpu.SemaphoreType.DMA,
    ],
)
def _sc_agg(y_hbm, src_hbm, dst_hbm, out_hbm, sidx_v, didx_v, rows_v, agg_sh,
            sem):
    c = lax.axis_index("c")
    s = lax.axis_index("s")
    wid = c * NS + s

    _zero_f32_buf(rows_v, CH, D)
    for r in range(RPT // CH):
        pltpu.sync_copy(rows_v, agg_sh.at[pl.ds(s * RPT + r * CH, CH)])
    plsc.subcore_barrier()

    @pl.loop(0, NCH)
    def _(j):
        base = wid * EPW + j * CH
        pltpu.sync_copy(src_hbm.at[pl.ds(base, CH)], sidx_v)
        pltpu.async_copy(y_hbm.at[sidx_v], rows_v, sem).wait()
        pltpu.sync_copy(dst_hbm.at[pl.ds(base, CH)], didx_v)
        pltpu.sync_copy(rows_v, agg_sh.at[didx_v], add=True)

    plsc.subcore_barrier()
    pltpu.sync_copy(agg_sh.at[pl.ds(s * RPT, RPT)],
                    out_hbm.at[c, pl.ds(s * RPT, RPT)])


def _dis_from_deg(deg_ref):
    cnt = deg_ref[0, :, 0:1] + deg_ref[1, :, 0:1]
    return lax.rsqrt(cnt + 2.0)


def _tc_first_body(deg_ref, x_ref, w_ref, y_ref):
    dis = _dis_from_deg(deg_ref)
    y_ref[...] = dis * jnp.dot(x_ref[...], w_ref[...],
                               preferred_element_type=jnp.float32)


def _bn_relu(t, g_ref, be_ref):
    tv = t[:N]
    mean = jnp.mean(tv, axis=0, keepdims=True)
    var = jnp.mean((tv - mean) ** 2, axis=0, keepdims=True)
    h = (t - mean) * lax.rsqrt(var + EPS) * g_ref[...] + be_ref[...]
    return jnp.maximum(h, 0.0)


def _tc_mid_body(deg_ref, agg_ref, y_ref, b_ref, g_ref, be_ref, w_ref,
                 out_ref):
    dis = _dis_from_deg(deg_ref)
    t = dis * (agg_ref[0] + agg_ref[1] + 2.0 * y_ref[...]) + b_ref[...]
    h = _bn_relu(t, g_ref, be_ref)
    rows = lax.broadcasted_iota(jnp.int32, (NP, 1), 0)
    h = jnp.where(rows < N, h, 0.0)
    out_ref[...] = dis * jnp.dot(h, w_ref[...],
                                 preferred_element_type=jnp.float32)


def _tc_last_body(deg_ref, agg_ref, y_ref, b_ref, g_ref, be_ref, out_ref):
    dis = _dis_from_deg(deg_ref)
    t = dis * (agg_ref[0] + agg_ref[1] + 2.0 * y_ref[...]) + b_ref[...]
    out_ref[...] = _bn_relu(t, g_ref, be_ref)


_f32 = jnp.float32
_tc_first = pl.pallas_call(
    _tc_first_body, out_shape=jax.ShapeDtypeStruct((NP, D), _f32))
_tc_mid = pl.pallas_call(
    _tc_mid_body, out_shape=jax.ShapeDtypeStruct((NP, D), _f32))
_tc_last = pl.pallas_call(
    _tc_last_body, out_shape=jax.ShapeDtypeStruct((NP, D), _f32))


def kernel(x, edge_index, W0, b0, g0, be0, W1, b1, g1, be1, W2, b2, g2, be2):
    src = edge_index[0].astype(jnp.int32)
    dst = edge_index[1].astype(jnp.int32)
    xp = jnp.pad(x, ((0, NP - N), (0, 0)))

    degp = _sc_deg(dst)
    y = _tc_first(degp, xp, W0)

    layers = ((b0, g0, be0, W1), (b1, g1, be1, W2), (b2, g2, be2, None))
    h = None
    for b, g, be, Wn in layers:
        aggp = _sc_agg(y, src, dst)
        b2d = b.reshape(1, D)
        g2d = g.reshape(1, D)
        be2d = be.reshape(1, D)
        if Wn is not None:
            y = _tc_mid(degp, aggp, y, b2d, g2d, be2d, Wn)
        else:
            h = _tc_last(degp, aggp, y, b2d, g2d, be2d)
    return h[:N]


# speedup vs baseline: 10.9295x; 10.9295x over previous
import functools

import jax
import jax.numpy as jnp
from jax import lax
from jax.experimental import pallas as pl
from jax.experimental.pallas import tpu as pltpu
from jax.experimental.pallas import tpu_sc as plsc

N = 10000
NP = 10240
D = 128
E = 320000
EPS = 1e-5

NC = 2
NS = 16
NW = NC * NS
EPW = E // NW
CH = 80
NCH = EPW // CH
RPT = NP // NS

_mesh = plsc.VectorSubcoreMesh(core_axis_name="c", subcore_axis_name="s")


def _zero_f32_buf(ref, rows, cols):
    @pl.loop(0, rows)
    def _(i):
        for j in range(cols // 16):
            ref[i, pl.ds(j * 16, 16)] = jnp.zeros((16,), jnp.float32)


@functools.partial(
    pl.kernel,
    out_type=jax.ShapeDtypeStruct((NC, NP, 16), jnp.float32),
    mesh=_mesh,
    scratch_types=[
        pltpu.VMEM((CH,), jnp.int32),
        pltpu.VMEM((CH,), jnp.int32),
        pltpu.VMEM((CH, 16), jnp.float32),
        pltpu.VMEM((CH, 16), jnp.float32),
        pltpu.VMEM_SHARED((NP, 16), jnp.float32),
    ],
)
def _sc_deg(dst_hbm, rows_hbm, out_hbm, didx_v, ridx_v, ones_v, zbuf_v,
            deg_sh):
    c = lax.axis_index("c")
    s = lax.axis_index("s")
    wid = c * NS + s

    @pl.loop(0, CH)
    def _(i):
        ones_v[i, pl.ds(0, 16)] = jnp.ones((16,), jnp.float32)

    _zero_f32_buf(zbuf_v, CH, 16)
    for r in range(RPT // CH):
        pltpu.sync_copy(rows_hbm.at[pl.ds(s * RPT + r * CH, CH)], ridx_v)
        pltpu.sync_copy(zbuf_v, deg_sh.at[ridx_v])
    plsc.subcore_barrier()

    @pl.loop(0, NCH)
    def _(j):
        base = wid * EPW + j * CH
        pltpu.sync_copy(dst_hbm.at[pl.ds(base, CH)], didx_v)
        pltpu.sync_copy(ones_v, deg_sh.at[didx_v], add=True)

    plsc.subcore_barrier()
    for r in range(RPT // CH):
        pltpu.sync_copy(rows_hbm.at[pl.ds(s * RPT + r * CH, CH)], ridx_v)
        pltpu.sync_copy(deg_sh.at[ridx_v], zbuf_v)
        pltpu.sync_copy(zbuf_v, out_hbm.at[c, pl.ds(s * RPT + r * CH, CH)])


@functools.partial(
    pl.kernel,
    out_type=jax.ShapeDtypeStruct((NC, NP, D), jnp.float32),
    mesh=_mesh,
    scratch_types=[
        pltpu.VMEM((CH,), jnp.int32),
        pltpu.VMEM((CH,), jnp.int32),
        pltpu.VMEM((CH,), jnp.int32),
        pltpu.VMEM((CH, D), jnp.float32),
        pltpu.VMEM_SHARED((NP, D), jnp.float32),
        pltpu.SemaphoreType.DMA,
    ],
)
def _sc_agg(y_hbm, src_hbm, dst_hbm, rows_hbm, out_hbm, sidx_v, didx_v,
            ridx_v, rows_v, agg_sh, sem):
    c = lax.axis_index("c")
    s = lax.axis_index("s")
    wid = c * NS + s

    _zero_f32_buf(rows_v, CH, D)
    for r in range(RPT // CH):
        pltpu.sync_copy(rows_hbm.at[pl.ds(s * RPT + r * CH, CH)], ridx_v)
        pltpu.sync_copy(rows_v, agg_sh.at[ridx_v])
    plsc.subcore_barrier()

    @pl.loop(0, NCH)
    def _(j):
        base = wid * EPW + j * CH
        pltpu.sync_copy(src_hbm.at[pl.ds(base, CH)], sidx_v)
        pltpu.async_copy(y_hbm.at[sidx_v], rows_v, sem).wait()
        pltpu.sync_copy(dst_hbm.at[pl.ds(base, CH)], didx_v)
        pltpu.sync_copy(rows_v, agg_sh.at[didx_v], add=True)

    plsc.subcore_barrier()
    for r in range(RPT // CH):
        pltpu.sync_copy(rows_hbm.at[pl.ds(s * RPT + r * CH, CH)], ridx_v)
        pltpu.sync_copy(agg_sh.at[ridx_v], rows_v)
        pltpu.sync_copy(rows_v, out_hbm.at[c, pl.ds(s * RPT + r * CH, CH)])


def _dis_from_deg(deg_ref):
    cnt = deg_ref[0, :, 0:1] + deg_ref[1, :, 0:1]
    return lax.rsqrt(cnt + 2.0)


def _tc_first_body(deg_ref, x_ref, w_ref, y_ref):
    dis = _dis_from_deg(deg_ref)
    y_ref[...] = dis * jnp.dot(x_ref[...], w_ref[...],
                               preferred_element_type=jnp.float32)


def _bn_relu(t, g_ref, be_ref):
    tv = t[:N]
    mean = jnp.mean(tv, axis=0, keepdims=True)
    var = jnp.mean((tv - mean) ** 2, axis=0, keepdims=True)
    h = (t - mean) * lax.rsqrt(var + EPS) * g_ref[...] + be_ref[...]
    return jnp.maximum(h, 0.0)


def _tc_mid_body(deg_ref, agg_ref, y_ref, b_ref, g_ref, be_ref, w_ref,
                 out_ref):
    dis = _dis_from_deg(deg_ref)
    t = dis * (agg_ref[0] + agg_ref[1] + 2.0 * y_ref[...]) + b_ref[...]
    h = _bn_relu(t, g_ref, be_ref)
    rows = lax.broadcasted_iota(jnp.int32, (NP, 1), 0)
    h = jnp.where(rows < N, h, 0.0)
    out_ref[...] = dis * jnp.dot(h, w_ref[...],
                                 preferred_element_type=jnp.float32)


def _tc_last_body(deg_ref, agg_ref, y_ref, b_ref, g_ref, be_ref, out_ref):
    dis = _dis_from_deg(deg_ref)
    t = dis * (agg_ref[0] + agg_ref[1] + 2.0 * y_ref[...]) + b_ref[...]
    out_ref[...] = _bn_relu(t, g_ref, be_ref)


_f32 = jnp.float32
_tc_first = pl.pallas_call(
    _tc_first_body, out_shape=jax.ShapeDtypeStruct((NP, D), _f32))
_tc_mid = pl.pallas_call(
    _tc_mid_body, out_shape=jax.ShapeDtypeStruct((NP, D), _f32))
_tc_last = pl.pallas_call(
    _tc_last_body, out_shape=jax.ShapeDtypeStruct((NP, D), _f32))


def _dbg_deg(dst):
    cnt = jnp.zeros((NP,), jnp.float32).at[dst].add(1.0)
    degp = jnp.zeros((NC, NP, 16), jnp.float32)
    return degp.at[0].set(cnt[:, None])


def _dbg_agg(y, src, dst):
    agg = jnp.zeros((NP, D), jnp.float32).at[dst].add(y[src])
    aggp = jnp.zeros((NC, NP, D), jnp.float32)
    return aggp.at[0].set(agg)


def kernel(x, edge_index, W0, b0, g0, be0, W1, b1, g1, be1, W2, b2, g2, be2):
    src = edge_index[0].astype(jnp.int32)
    dst = edge_index[1].astype(jnp.int32)
    xp = jnp.pad(x, ((0, NP - N), (0, 0)))
    rowids = jnp.arange(NP, dtype=jnp.int32)

    degp = _sc_deg(dst, rowids)
    y = _tc_first(degp, xp, W0)

    layers = ((b0, g0, be0, W1), (b1, g1, be1, W2), (b2, g2, be2, None))
    h = None
    for b, g, be, Wn in layers:
        aggp = _sc_agg(y, src, dst, rowids)
        b2d = b.reshape(1, D)
        g2d = g.reshape(1, D)
        be2d = be.reshape(1, D)
        if Wn is not None:
            y = _tc_mid(degp, aggp, y, b2d, g2d, be2d, Wn)
        else:
            h = _tc_last(degp, aggp, y, b2d, g2d, be2d)
    return h[:N]


# speedup vs baseline: 15.7606x; 1.4420x over previous
import functools

import jax
import jax.numpy as jnp
from jax import lax
from jax.experimental import pallas as pl
from jax.experimental.pallas import tpu as pltpu
from jax.experimental.pallas import tpu_sc as plsc

N = 10000
NP = 10240
D = 128
E = 320000
EPS = 1e-5

NC = 2
NS = 16
NW = NC * NS
EPW = E // NW
CH = 80
NCH = EPW // CH
RPT = NP // NS

_mesh = plsc.VectorSubcoreMesh(core_axis_name="c", subcore_axis_name="s")


def _zero_f32_buf(ref, rows, cols):
    @pl.loop(0, rows)
    def _(i):
        for j in range(cols // 16):
            ref[i, pl.ds(j * 16, 16)] = jnp.zeros((16,), jnp.float32)


@functools.partial(
    pl.kernel,
    out_type=jax.ShapeDtypeStruct((NC, NP, 16), jnp.float32),
    mesh=_mesh,
    scratch_types=[
        pltpu.VMEM((CH,), jnp.int32),
        pltpu.VMEM((CH,), jnp.int32),
        pltpu.VMEM((CH, 16), jnp.float32),
        pltpu.VMEM((CH, 16), jnp.float32),
        pltpu.VMEM_SHARED((NP, 16), jnp.float32),
    ],
)
def _sc_deg(dst_hbm, rows_hbm, out_hbm, didx_v, ridx_v, ones_v, zbuf_v,
            deg_sh):
    c = lax.axis_index("c")
    s = lax.axis_index("s")
    wid = c * NS + s

    @pl.loop(0, CH)
    def _(i):
        ones_v[i, pl.ds(0, 16)] = jnp.ones((16,), jnp.float32)

    _zero_f32_buf(zbuf_v, CH, 16)
    for r in range(RPT // CH):
        pltpu.sync_copy(rows_hbm.at[pl.ds(s * RPT + r * CH, CH)], ridx_v)
        pltpu.sync_copy(zbuf_v, deg_sh.at[ridx_v])
    plsc.subcore_barrier()

    @pl.loop(0, NCH)
    def _(j):
        base = wid * EPW + j * CH
        pltpu.sync_copy(dst_hbm.at[pl.ds(base, CH)], didx_v)
        pltpu.sync_copy(ones_v, deg_sh.at[didx_v], add=True)

    plsc.subcore_barrier()
    for r in range(RPT // CH):
        pltpu.sync_copy(rows_hbm.at[pl.ds(s * RPT + r * CH, CH)], ridx_v)
        pltpu.sync_copy(deg_sh.at[ridx_v], zbuf_v)
        pltpu.sync_copy(zbuf_v, out_hbm.at[c, pl.ds(s * RPT + r * CH, CH)])


@functools.partial(
    pl.kernel,
    out_type=jax.ShapeDtypeStruct((NC, NP, D), jnp.float32),
    mesh=_mesh,
    scratch_types=[
        pltpu.VMEM((CH,), jnp.int32),
        pltpu.VMEM((CH,), jnp.int32),
        pltpu.VMEM((CH,), jnp.int32),
        pltpu.VMEM((CH,), jnp.int32),
        pltpu.VMEM((CH,), jnp.int32),
        pltpu.VMEM((CH, D), jnp.float32),
        pltpu.VMEM((CH, D), jnp.float32),
        pltpu.VMEM_SHARED((NP, D), jnp.float32),
        pltpu.SemaphoreType.DMA,
        pltpu.SemaphoreType.DMA,
    ],
)
def _sc_agg(y_hbm, src_hbm, dst_hbm, rows_hbm, out_hbm, sidx_a, sidx_b,
            didx_a, didx_b, ridx_v, rows_a, rows_b, agg_sh, sem_a, sem_b):
    c = lax.axis_index("c")
    s = lax.axis_index("s")
    wid = c * NS + s

    _zero_f32_buf(rows_a, CH, D)
    for r in range(RPT // CH):
        pltpu.sync_copy(rows_hbm.at[pl.ds(s * RPT + r * CH, CH)], ridx_v)
        pltpu.sync_copy(rows_a, agg_sh.at[ridx_v])
    plsc.subcore_barrier()

    @pl.loop(0, NCH // 2)
    def _(i):
        base_a = wid * EPW + (2 * i) * CH
        base_b = base_a + CH
        pltpu.sync_copy(src_hbm.at[pl.ds(base_a, CH)], sidx_a)
        ga = pltpu.async_copy(y_hbm.at[sidx_a], rows_a, sem_a)
        pltpu.sync_copy(src_hbm.at[pl.ds(base_b, CH)], sidx_b)
        gb = pltpu.async_copy(y_hbm.at[sidx_b], rows_b, sem_a)
        pltpu.sync_copy(dst_hbm.at[pl.ds(base_a, CH)], didx_a)
        pltpu.sync_copy(dst_hbm.at[pl.ds(base_b, CH)], didx_b)
        ga.wait()
        gb.wait()
        pltpu.sync_copy(rows_a, agg_sh.at[didx_a], add=True)
        pltpu.sync_copy(rows_b, agg_sh.at[didx_b], add=True)

    base_t = wid * EPW + (NCH - 1) * CH
    pltpu.sync_copy(src_hbm.at[pl.ds(base_t, CH)], sidx_a)
    pltpu.async_copy(y_hbm.at[sidx_a], rows_a, sem_a).wait()
    pltpu.sync_copy(dst_hbm.at[pl.ds(base_t, CH)], didx_a)
    pltpu.sync_copy(rows_a, agg_sh.at[didx_a], add=True)

    plsc.subcore_barrier()
    for r in range(RPT // CH):
        pltpu.sync_copy(rows_hbm.at[pl.ds(s * RPT + r * CH, CH)], ridx_v)
        pltpu.sync_copy(agg_sh.at[ridx_v], rows_a)
        pltpu.sync_copy(rows_a, out_hbm.at[c, pl.ds(s * RPT + r * CH, CH)])


def _dis_from_deg(deg_ref):
    cnt = deg_ref[0, :, 0:1] + deg_ref[1, :, 0:1]
    return lax.rsqrt(cnt + 2.0)


def _tc_first_body(deg_ref, x_ref, w_ref, y_ref):
    dis = _dis_from_deg(deg_ref)
    y_ref[...] = dis * jnp.dot(x_ref[...], w_ref[...],
                               preferred_element_type=jnp.float32)


def _bn_relu(t, g_ref, be_ref):
    tv = t[:N]
    mean = jnp.mean(tv, axis=0, keepdims=True)
    var = jnp.mean((tv - mean) ** 2, axis=0, keepdims=True)
    h = (t - mean) * lax.rsqrt(var + EPS) * g_ref[...] + be_ref[...]
    return jnp.maximum(h, 0.0)


def _tc_mid_body(deg_ref, agg_ref, y_ref, b_ref, g_ref, be_ref, w_ref,
                 out_ref):
    dis = _dis_from_deg(deg_ref)
    t = dis * (agg_ref[0] + agg_ref[1] + 2.0 * y_ref[...]) + b_ref[...]
    h = _bn_relu(t, g_ref, be_ref)
    rows = lax.broadcasted_iota(jnp.int32, (NP, 1), 0)
    h = jnp.where(rows < N, h, 0.0)
    out_ref[...] = dis * jnp.dot(h, w_ref[...],
                                 preferred_element_type=jnp.float32)


def _tc_last_body(deg_ref, agg_ref, y_ref, b_ref, g_ref, be_ref, out_ref):
    dis = _dis_from_deg(deg_ref)
    t = dis * (agg_ref[0] + agg_ref[1] + 2.0 * y_ref[...]) + b_ref[...]
    out_ref[...] = _bn_relu(t, g_ref, be_ref)


_f32 = jnp.float32
_tc_first = pl.pallas_call(
    _tc_first_body, out_shape=jax.ShapeDtypeStruct((NP, D), _f32))
_tc_mid = pl.pallas_call(
    _tc_mid_body, out_shape=jax.ShapeDtypeStruct((NP, D), _f32))
_tc_last = pl.pallas_call(
    _tc_last_body, out_shape=jax.ShapeDtypeStruct((NP, D), _f32))


def _dbg_deg(dst):
    cnt = jnp.zeros((NP,), jnp.float32).at[dst].add(1.0)
    degp = jnp.zeros((NC, NP, 16), jnp.float32)
    return degp.at[0].set(cnt[:, None])


def _dbg_agg(y, src, dst):
    agg = jnp.zeros((NP, D), jnp.float32).at[dst].add(y[src])
    aggp = jnp.zeros((NC, NP, D), jnp.float32)
    return aggp.at[0].set(agg)


def kernel(x, edge_index, W0, b0, g0, be0, W1, b1, g1, be1, W2, b2, g2, be2):
    src = edge_index[0].astype(jnp.int32)
    dst = edge_index[1].astype(jnp.int32)
    xp = jnp.pad(x, ((0, NP - N), (0, 0)))
    rowids = jnp.arange(NP, dtype=jnp.int32)

    degp = _sc_deg(dst, rowids)
    y = _tc_first(degp, xp, W0)

    layers = ((b0, g0, be0, W1), (b1, g1, be1, W2), (b2, g2, be2, None))
    h = None
    for b, g, be, Wn in layers:
        aggp = _sc_agg(y, src, dst, rowids)
        b2d = b.reshape(1, D)
        g2d = g.reshape(1, D)
        be2d = be.reshape(1, D)
        if Wn is not None:
            y = _tc_mid(degp, aggp, y, b2d, g2d, be2d, Wn)
        else:
            h = _tc_last(degp, aggp, y, b2d, g2d, be2d)
    return h[:N]


# speedup vs baseline: 16.3380x; 1.0366x over previous
import functools

import jax
import jax.numpy as jnp
from jax import lax
from jax.experimental import pallas as pl
from jax.experimental.pallas import tpu as pltpu
from jax.experimental.pallas import tpu_sc as plsc

N = 10000
NP = 10240
D = 128
E = 320000
EPS = 1e-5

NC = 2
NS = 16
NW = NC * NS
EPW = E // NW
CH = 80
NCH = EPW // CH
RPT = NP // NS

_mesh = plsc.VectorSubcoreMesh(core_axis_name="c", subcore_axis_name="s")


def _zero_f32_buf(ref, rows, cols):
    @pl.loop(0, rows)
    def _(i):
        for j in range(cols // 16):
            ref[i, pl.ds(j * 16, 16)] = jnp.zeros((16,), jnp.float32)


@functools.partial(
    pl.kernel,
    out_type=jax.ShapeDtypeStruct((NC, NP, 16), jnp.float32),
    mesh=_mesh,
    scratch_types=[
        pltpu.VMEM((CH,), jnp.int32),
        pltpu.VMEM((CH,), jnp.int32),
        pltpu.VMEM((CH, 16), jnp.float32),
        pltpu.VMEM((CH, 16), jnp.float32),
        pltpu.VMEM_SHARED((NP, 16), jnp.float32),
    ],
)
def _sc_deg(dst_hbm, rows_hbm, out_hbm, didx_v, ridx_v, ones_v, zbuf_v,
            deg_sh):
    c = lax.axis_index("c")
    s = lax.axis_index("s")
    wid = c * NS + s

    @pl.loop(0, CH)
    def _(i):
        ones_v[i, pl.ds(0, 16)] = jnp.ones((16,), jnp.float32)

    _zero_f32_buf(zbuf_v, CH, 16)
    for r in range(RPT // CH):
        pltpu.sync_copy(rows_hbm.at[pl.ds(s * RPT + r * CH, CH)], ridx_v)
        pltpu.sync_copy(zbuf_v, deg_sh.at[ridx_v])
    plsc.subcore_barrier()

    @pl.loop(0, NCH)
    def _(j):
        base = wid * EPW + j * CH
        pltpu.sync_copy(dst_hbm.at[pl.ds(base, CH)], didx_v)
        pltpu.sync_copy(ones_v, deg_sh.at[didx_v], add=True)

    plsc.subcore_barrier()
    for r in range(RPT // CH):
        pltpu.sync_copy(rows_hbm.at[pl.ds(s * RPT + r * CH, CH)], ridx_v)
        pltpu.sync_copy(deg_sh.at[ridx_v], zbuf_v)
        pltpu.sync_copy(zbuf_v, out_hbm.at[c, pl.ds(s * RPT + r * CH, CH)])


NBUF = 3


@functools.partial(
    pl.kernel,
    out_type=jax.ShapeDtypeStruct((NC, NP, D), jnp.float32),
    mesh=_mesh,
    scratch_types=(
        [pltpu.VMEM((CH,), jnp.int32) for _ in range(NBUF)]
        + [pltpu.VMEM((CH,), jnp.int32) for _ in range(NBUF)]
        + [pltpu.VMEM((CH,), jnp.int32)]
        + [pltpu.VMEM((CH, D), jnp.float32) for _ in range(NBUF)]
        + [pltpu.VMEM_SHARED((NP, D), jnp.float32),
           pltpu.SemaphoreType.DMA]
    ),
)
def _sc_agg(y_hbm, src_hbm, dst_hbm, rows_hbm, out_hbm, *scr):
    sidx = scr[0:NBUF]
    didx = scr[NBUF:2 * NBUF]
    ridx_v = scr[2 * NBUF]
    rows = scr[2 * NBUF + 1:3 * NBUF + 1]
    agg_sh = scr[3 * NBUF + 1]
    sem = scr[3 * NBUF + 2]
    c = lax.axis_index("c")
    s = lax.axis_index("s")
    wid = c * NS + s

    _zero_f32_buf(rows[0], CH, D)
    for r in range(RPT // CH):
        pltpu.sync_copy(rows_hbm.at[pl.ds(s * RPT + r * CH, CH)], ridx_v)
        pltpu.sync_copy(rows[0], agg_sh.at[ridx_v])
    plsc.subcore_barrier()

    @pl.loop(0, NCH // NBUF)
    def _(i):
        base = wid * EPW + (NBUF * i) * CH
        gs = []
        for k in range(NBUF):
            pltpu.sync_copy(src_hbm.at[pl.ds(base + k * CH, CH)], sidx[k])
            gs.append(pltpu.async_copy(y_hbm.at[sidx[k]], rows[k], sem))
        for k in range(NBUF):
            pltpu.sync_copy(dst_hbm.at[pl.ds(base + k * CH, CH)], didx[k])
        for g in gs:
            g.wait()
        for k in range(NBUF):
            pltpu.sync_copy(rows[k], agg_sh.at[didx[k]], add=True)

    for j in range(NCH - (NCH // NBUF) * NBUF):
        base_t = wid * EPW + ((NCH // NBUF) * NBUF + j) * CH
        pltpu.sync_copy(src_hbm.at[pl.ds(base_t, CH)], sidx[0])
        pltpu.async_copy(y_hbm.at[sidx[0]], rows[0], sem).wait()
        pltpu.sync_copy(dst_hbm.at[pl.ds(base_t, CH)], didx[0])
        pltpu.sync_copy(rows[0], agg_sh.at[didx[0]], add=True)

    plsc.subcore_barrier()
    for r in range(RPT // CH):
        pltpu.sync_copy(rows_hbm.at[pl.ds(s * RPT + r * CH, CH)], ridx_v)
        pltpu.sync_copy(agg_sh.at[ridx_v], rows[0])
        pltpu.sync_copy(rows[0], out_hbm.at[c, pl.ds(s * RPT + r * CH, CH)])


def _dis_from_deg(deg_ref):
    cnt = deg_ref[0, :, 0:1] + deg_ref[1, :, 0:1]
    return lax.rsqrt(cnt + 2.0)


def _tc_first_body(deg_ref, x_ref, w_ref, y_ref):
    dis = _dis_from_deg(deg_ref)
    y_ref[...] = dis * jnp.dot(x_ref[...], w_ref[...],
                               preferred_element_type=jnp.float32)


def _bn_relu(t, g_ref, be_ref):
    tv = t[:N]
    mean = jnp.mean(tv, axis=0, keepdims=True)
    var = jnp.mean((tv - mean) ** 2, axis=0, keepdims=True)
    h = (t - mean) * lax.rsqrt(var + EPS) * g_ref[...] + be_ref[...]
    return jnp.maximum(h, 0.0)


def _tc_mid_body(deg_ref, agg_ref, y_ref, b_ref, g_ref, be_ref, w_ref,
                 out_ref):
    dis = _dis_from_deg(deg_ref)
    t = dis * (agg_ref[0] + agg_ref[1] + 2.0 * y_ref[...]) + b_ref[...]
    h = _bn_relu(t, g_ref, be_ref)
    rows = lax.broadcasted_iota(jnp.int32, (NP, 1), 0)
    h = jnp.where(rows < N, h, 0.0)
    out_ref[...] = dis * jnp.dot(h, w_ref[...],
                                 preferred_element_type=jnp.float32)


def _tc_last_body(deg_ref, agg_ref, y_ref, b_ref, g_ref, be_ref, out_ref):
    dis = _dis_from_deg(deg_ref)
    t = dis * (agg_ref[0] + agg_ref[1] + 2.0 * y_ref[...]) + b_ref[...]
    out_ref[...] = _bn_relu(t, g_ref, be_ref)


_f32 = jnp.float32
_tc_first = pl.pallas_call(
    _tc_first_body, out_shape=jax.ShapeDtypeStruct((NP, D), _f32))
_tc_mid = pl.pallas_call(
    _tc_mid_body, out_shape=jax.ShapeDtypeStruct((NP, D), _f32))
_tc_last = pl.pallas_call(
    _tc_last_body, out_shape=jax.ShapeDtypeStruct((NP, D), _f32))


def _dbg_deg(dst):
    cnt = jnp.zeros((NP,), jnp.float32).at[dst].add(1.0)
    degp = jnp.zeros((NC, NP, 16), jnp.float32)
    return degp.at[0].set(cnt[:, None])


def _dbg_agg(y, src, dst):
    agg = jnp.zeros((NP, D), jnp.float32).at[dst].add(y[src])
    aggp = jnp.zeros((NC, NP, D), jnp.float32)
    return aggp.at[0].set(agg)


def kernel(x, edge_index, W0, b0, g0, be0, W1, b1, g1, be1, W2, b2, g2, be2):
    src = edge_index[0].astype(jnp.int32)
    dst = edge_index[1].astype(jnp.int32)
    xp = jnp.pad(x, ((0, NP - N), (0, 0)))
    rowids = jnp.arange(NP, dtype=jnp.int32)

    degp = _sc_deg(dst, rowids)
    y = _tc_first(degp, xp, W0)

    layers = ((b0, g0, be0, W1), (b1, g1, be1, W2), (b2, g2, be2, None))
    h = None
    for b, g, be, Wn in layers:
        aggp = _sc_agg(y, src, dst, rowids)
        b2d = b.reshape(1, D)
        g2d = g.reshape(1, D)
        be2d = be.reshape(1, D)
        if Wn is not None:
            y = _tc_mid(degp, aggp, y, b2d, g2d, be2d, Wn)
        else:
            h = _tc_last(degp, aggp, y, b2d, g2d, be2d)
    return h[:N]
